# Initial kernel scaffold; baseline (speedup 1.0000x reference)
import jax
import jax.numpy as jnp
from jax.experimental import pallas as pl


def kernel(sub_goal, seq_length, pos_table, gamma, beta):
    raise NotImplementedError("write your pallas kernel here")


# speedup vs baseline: 1.5420x; 1.5420x over previous
import functools

import jax
import jax.numpy as jnp
from jax.experimental import pallas as pl

_HID = 768
_BS = 512


def _ln_body(sub_ref, pos_ref, gamma_ref, beta_ref, out_ref):
    x = sub_ref[0]
    p = pos_ref[...]
    e = p + x
    mean = jnp.mean(e, axis=-1, keepdims=True)
    c = e - mean
    var = jnp.mean(c * c, axis=-1, keepdims=True)
    r = jax.lax.rsqrt(var + 1e-12)
    out_ref[0] = (c * r) * gamma_ref[...] + beta_ref[...]


@functools.partial(jax.jit, static_argnums=())
def _run(sub2d, table, gamma2d, beta2d):
    S = table.shape[0]
    BN = sub2d.shape[0]
    sub3d = sub2d.reshape(BN, 1, _HID)
    grid = (S // _BS, BN)
    out = pl.pallas_call(
        _ln_body,
        grid=grid,
        in_specs=[
            pl.BlockSpec((1, 1, _HID), lambda i, j: (j, 0, 0)),
            pl.BlockSpec((_BS, _HID), lambda i, j: (i, 0)),
            pl.BlockSpec((1, _HID), lambda i, j: (0, 0)),
            pl.BlockSpec((1, _HID), lambda i, j: (0, 0)),
        ],
        out_specs=pl.BlockSpec((1, _BS, _HID), lambda i, j: (j, i, 0)),
        out_shape=jax.ShapeDtypeStruct((BN, S, _HID), jnp.float32),
    )(sub3d, table, gamma2d, beta2d)
    return out


def kernel(sub_goal, seq_length, pos_table, gamma, beta):
    B, N, H = sub_goal.shape
    S = pos_table.shape[0]
    sub2d = sub_goal.reshape(B * N, H)
    out = _run(sub2d, pos_table, gamma.reshape(1, H), beta.reshape(1, H))
    return out.reshape(B, N, S, H)
